# baseline (speedup 1.0000x reference)
import functools

import jax
import jax.numpy as jnp
from jax import lax
from jax.experimental import pallas as pl
from jax.experimental.pallas import tpu as pltpu
from jax.experimental.pallas import tpu_sc as plsc

N = 50000
E = 800000
HID = 64
LAT = 32

NP = 51200
EP = 802816
TOT = float(E + N)
EPS = 1e-5

CHUNK = 128
GC = 64
NTILE = 16
NCORE = 2

def _sc_mesh():
    return plsc.VectorSubcoreMesh(core_axis_name="c", subcore_axis_name="s")

RE = 4096
RN = 6400

_f32 = jnp.float32


def _masked_bn_relu(z, mask, count):
    zm = jnp.where(mask, z, 0.0)
    s1 = jnp.sum(zm, axis=0)
    s2 = jnp.sum(zm * zm, axis=0)
    m = s1 / count
    v = s2 / count - m * m
    inv = lax.rsqrt(v + EPS)
    return jnp.maximum((z - m) * inv, 0.0)


def _mlp_step_body(x_ref, stin_ref, w_ref, b_ref, out_ref, stout_ref, acc_ref,
                   *, norm, mask_out):
    j = pl.program_id(0)

    @pl.when(j == 0)
    def _():
        acc_ref[...] = jnp.zeros_like(acc_ref)

    x = x_ref[...]
    if norm:
        x = jnp.maximum((x - stin_ref[0:1, :]) * stin_ref[1:2, :], 0.0)
    z = jnp.dot(x, w_ref[...], preferred_element_type=_f32) + b_ref[...]
    rows = lax.broadcasted_iota(jnp.int32, (RN, 1), 0) + j * RN
    mask = rows < N
    zm = jnp.where(mask, z, 0.0)
    out_ref[...] = zm if mask_out else z
    acc_ref[0:1, :] += jnp.sum(zm, axis=0, keepdims=True)
    acc_ref[1:2, :] += jnp.sum(zm * zm, axis=0, keepdims=True)

    @pl.when(j == pl.num_programs(0) - 1)
    def _():
        m = acc_ref[0:1, :] / N
        v = acc_ref[1:2, :] / N - m * m
        stout_ref[...] = jnp.concatenate(
            [m, lax.rsqrt(v + EPS),
             jnp.zeros((6, acc_ref.shape[1]), _f32)], axis=0)


def _mlp_step(x, stin, w, b, *, norm, mask_out=False):
    din = x.shape[1]
    dout = w.shape[1]
    return pl.pallas_call(
        functools.partial(_mlp_step_body, norm=norm, mask_out=mask_out),
        grid=(NP // RN,),
        in_specs=[
            pl.BlockSpec((RN, din), lambda j: (j, 0)),
            pl.BlockSpec((8, din), lambda j: (0, 0)),
            pl.BlockSpec((din, dout), lambda j: (0, 0)),
            pl.BlockSpec((1, dout), lambda j: (0, 0)),
        ],
        out_specs=[
            pl.BlockSpec((RN, dout), lambda j: (j, 0)),
            pl.BlockSpec((8, dout), lambda j: (0, 0)),
        ],
        out_shape=[jax.ShapeDtypeStruct((NP, dout), _f32),
                   jax.ShapeDtypeStruct((8, dout), _f32)],
        scratch_shapes=[pltpu.VMEM((8, dout), _f32)],
    )(x, stin, w, b)


def _node_mlp3(x, w0, b0, w1, b1, w2, b2):
    dummy = jnp.zeros((8, x.shape[1]), _f32)
    z1, st1 = _mlp_step(x, dummy, w0, b0, norm=False)
    z2, st2 = _mlp_step(z1, st1, w1, b1, norm=True)
    out, _ = _mlp_step(z2, st2, w2, b2, norm=True, mask_out=True)
    return out


def _edge_stats_body(ea_ref, w0_ref, b0_ref, st_ref, acc_ref):
    j = pl.program_id(0)

    @pl.when(j == 0)
    def _():
        acc_ref[...] = jnp.zeros_like(acc_ref)

    z = jnp.dot(ea_ref[...], w0_ref[...], preferred_element_type=_f32) + b0_ref[...]
    rows = lax.broadcasted_iota(jnp.int32, (RE, 1), 0) + j * RE
    zm = jnp.where(rows < E, z, 0.0)
    acc_ref[0:1, :] += jnp.sum(zm, axis=0, keepdims=True)
    acc_ref[1:2, :] += jnp.sum(zm * zm, axis=0, keepdims=True)

    @pl.when(j == pl.num_programs(0) - 1)
    def _():
        m = acc_ref[0:1, :] / E
        v = acc_ref[1:2, :] / E - m * m
        st_ref[...] = jnp.concatenate(
            [m, lax.rsqrt(v + EPS), jnp.zeros((6, HID), _f32)], axis=0)


def _edge_stats(eap, w0, b0):
    return pl.pallas_call(
        _edge_stats_body,
        grid=(EP // RE,),
        in_specs=[
            pl.BlockSpec((RE, 8), lambda j: (j, 0)),
            pl.BlockSpec((8, HID), lambda j: (0, 0)),
            pl.BlockSpec((1, HID), lambda j: (0, 0)),
        ],
        out_specs=pl.BlockSpec((8, HID), lambda j: (0, 0)),
        out_shape=jax.ShapeDtypeStruct((8, HID), _f32),
        scratch_shapes=[pltpu.VMEM((8, HID), _f32)],
    )(eap, w0, b0)


def _edge_latent_body(ea_ref, st_ref, w0_ref, b0_ref, w1_ref, b1_ref,
                      bcat_ref, b0c_ref,
                      eb0_ref, eb1_ref, eb2_ref, ebsum_ref, acc_ref):
    j = pl.program_id(0)

    @pl.when(j == 0)
    def _():
        acc_ref[...] = jnp.zeros_like(acc_ref)

    z = jnp.dot(ea_ref[...], w0_ref[...], preferred_element_type=_f32) + b0_ref[...]
    m = st_ref[0:1, :]
    inv = st_ref[1:2, :]
    zn = jnp.maximum((z - m) * inv, 0.0)
    el = jnp.dot(zn, w1_ref[...], preferred_element_type=_f32) + b1_ref[...]
    rows = lax.broadcasted_iota(jnp.int32, (RE, 1), 0) + j * RE
    mask = rows < E
    outs = (eb0_ref, eb1_ref, eb2_ref)
    for l in range(3):
        ebl = jnp.dot(el, bcat_ref[32 * l:32 * l + 32, :],
                      preferred_element_type=_f32) + b0c_ref[l:l + 1, :]
        ebl = jnp.where(mask, ebl, 0.0)
        outs[l][...] = ebl
        acc_ref[l:l + 1, :] += jnp.sum(ebl, axis=0, keepdims=True)

    @pl.when(j == pl.num_programs(0) - 1)
    def _():
        ebsum_ref[...] = acc_ref[...]


def _edge_latent(eap, est, w0, b0, w1, b1, bcat, b0cat):
    eb_sds = jax.ShapeDtypeStruct((EP, HID), _f32)
    return pl.pallas_call(
        _edge_latent_body,
        grid=(EP // RE,),
        in_specs=[
            pl.BlockSpec((RE, 8), lambda j: (j, 0)),
            pl.BlockSpec((8, HID), lambda j: (0, 0)),
            pl.BlockSpec((8, HID), lambda j: (0, 0)),
            pl.BlockSpec((1, HID), lambda j: (0, 0)),
            pl.BlockSpec((HID, LAT), lambda j: (0, 0)),
            pl.BlockSpec((1, LAT), lambda j: (0, 0)),
            pl.BlockSpec((96, HID), lambda j: (0, 0)),
            pl.BlockSpec((8, HID), lambda j: (0, 0)),
        ],
        out_specs=[
            pl.BlockSpec((RE, HID), lambda j: (j, 0)),
            pl.BlockSpec((RE, HID), lambda j: (j, 0)),
            pl.BlockSpec((RE, HID), lambda j: (j, 0)),
            pl.BlockSpec((8, HID), lambda j: (0, 0)),
        ],
        out_shape=[eb_sds, eb_sds, eb_sds,
                   jax.ShapeDtypeStruct((8, HID), _f32)],
        scratch_shapes=[pltpu.VMEM((8, HID), _f32)],
    )(eap, est, w0, b0, w1, b1, bcat, b0cat)


def _pre_body(h_ref, a_ref, c_ref, b_ref, b0_ref,
              gf_ref, zs_ref, ps_ref, acc_ref):
    j = pl.program_id(0)

    @pl.when(j == 0)
    def _():
        acc_ref[...] = jnp.zeros_like(acc_ref)

    h = h_ref[...]
    g = jnp.dot(h, a_ref[...], preferred_element_type=_f32)
    f = jnp.dot(h, c_ref[...], preferred_element_type=_f32)
    cb = jnp.sum(b_ref[...], axis=0, keepdims=True) + b0_ref[...]
    zs = g + f + cb
    gf_ref[...] = jnp.concatenate([g, f], axis=1)
    zs_ref[...] = zs
    rows = lax.broadcasted_iota(jnp.int32, (RN, 1), 0) + j * RN
    zsm = jnp.where(rows < N, zs, 0.0)
    acc_ref[0:1, :] += jnp.sum(zsm, axis=0, keepdims=True)
    acc_ref[1:2, :] += jnp.sum(zsm * zsm, axis=0, keepdims=True)

    @pl.when(j == pl.num_programs(0) - 1)
    def _():
        ps_ref[...] = acc_ref[...]


def _pre(h, a, c, b, b0):
    return pl.pallas_call(
        _pre_body,
        grid=(NP // RN,),
        in_specs=[
            pl.BlockSpec((RN, LAT), lambda j: (j, 0)),
            pl.BlockSpec((LAT, HID), lambda j: (0, 0)),
            pl.BlockSpec((LAT, HID), lambda j: (0, 0)),
            pl.BlockSpec((LAT, HID), lambda j: (0, 0)),
            pl.BlockSpec((1, HID), lambda j: (0, 0)),
        ],
        out_specs=[
            pl.BlockSpec((RN, 2 * HID), lambda j: (j, 0)),
            pl.BlockSpec((RN, HID), lambda j: (j, 0)),
            pl.BlockSpec((8, HID), lambda j: (0, 0)),
        ],
        out_shape=[jax.ShapeDtypeStruct((NP, 2 * HID), _f32),
                   jax.ShapeDtypeStruct((NP, HID), _f32),
                   jax.ShapeDtypeStruct((8, HID), _f32)],
        scratch_shapes=[pltpu.VMEM((8, HID), _f32)],
    )(h, a, c, b, b0)


def _fin_body(scst_ref, ps_ref, mv_ref):
    s = jnp.sum(scst_ref[...], axis=0, keepdims=True)
    s1 = s[:, :HID] + ps_ref[0:1, :]
    s2 = s[:, HID:] + ps_ref[1:2, :]
    m = s1 / TOT
    v = s2 / TOT - m * m
    mv_ref[...] = jnp.concatenate(
        [m, lax.rsqrt(v + EPS), jnp.zeros((6, HID), _f32)], axis=0)


def _fin(scst, ps):
    return pl.pallas_call(
        _fin_body,
        out_shape=jax.ShapeDtypeStruct((8, HID), _f32),
    )(scst, ps)


def _aggr_body(acca_ref, accb_ref, zs_ref, mv_ref, cnt_ref, w1_ref, b1_ref,
               out_ref):
    acca = acca_ref[...]
    accb = accb_ref[...]
    acc = jnp.concatenate([acca[0], acca[1], accb[0], accb[1]], axis=1)
    m = mv_ref[0:1, :]
    inv = mv_ref[1:2, :]
    full = acc + jnp.maximum(zs_ref[...] - m, 0.0)
    cnt = cnt_ref[...]
    c = cnt[0][:, 0:1] + cnt[1][:, 0:1] + 1.0
    out_ref[...] = (jnp.dot(full * inv, w1_ref[...],
                            preferred_element_type=_f32) / c) + b1_ref[...]


def _aggr(acca, accb, zs, mv, cnt, w1, b1):
    return pl.pallas_call(
        _aggr_body,
        grid=(NP // RN,),
        in_specs=[
            pl.BlockSpec((2, RN, 16), lambda j: (0, j, 0)),
            pl.BlockSpec((2, RN, 16), lambda j: (0, j, 0)),
            pl.BlockSpec((RN, HID), lambda j: (j, 0)),
            pl.BlockSpec((8, HID), lambda j: (0, 0)),
            pl.BlockSpec((2, RN, 16), lambda j: (0, j, 0)),
            pl.BlockSpec((HID, HID), lambda j: (0, 0)),
            pl.BlockSpec((1, HID), lambda j: (0, 0)),
        ],
        out_specs=pl.BlockSpec((RN, HID), lambda j: (j, 0)),
        out_shape=jax.ShapeDtypeStruct((NP, HID), _f32),
    )(acca, accb, zs, mv, cnt, w1, b1)


def _upd_stats_body(h_ref, ag_ref, u0a_ref, u0b_ref, b0_ref, st_ref, acc_ref):
    j = pl.program_id(0)

    @pl.when(j == 0)
    def _():
        acc_ref[...] = jnp.zeros_like(acc_ref)

    u = (jnp.dot(h_ref[...], u0a_ref[...], preferred_element_type=_f32)
         + jnp.dot(ag_ref[...], u0b_ref[...], preferred_element_type=_f32)
         + b0_ref[...])
    rows = lax.broadcasted_iota(jnp.int32, (RN, 1), 0) + j * RN
    um = jnp.where(rows < N, u, 0.0)
    acc_ref[0:1, :] += jnp.sum(um, axis=0, keepdims=True)
    acc_ref[1:2, :] += jnp.sum(um * um, axis=0, keepdims=True)

    @pl.when(j == pl.num_programs(0) - 1)
    def _():
        m = acc_ref[0:1, :] / N
        v = acc_ref[1:2, :] / N - m * m
        st_ref[...] = jnp.concatenate(
            [m, lax.rsqrt(v + EPS), jnp.zeros((6, HID), _f32)], axis=0)


def _upd_stats(h, aggr, u0a, u0b, b0):
    return pl.pallas_call(
        _upd_stats_body,
        grid=(NP // RN,),
        in_specs=[
            pl.BlockSpec((RN, LAT), lambda j: (j, 0)),
            pl.BlockSpec((RN, HID), lambda j: (j, 0)),
            pl.BlockSpec((LAT, HID), lambda j: (0, 0)),
            pl.BlockSpec((HID, HID), lambda j: (0, 0)),
            pl.BlockSpec((1, HID), lambda j: (0, 0)),
        ],
        out_specs=pl.BlockSpec((8, HID), lambda j: (0, 0)),
        out_shape=jax.ShapeDtypeStruct((8, HID), _f32),
        scratch_shapes=[pltpu.VMEM((8, HID), _f32)],
    )(h, aggr, u0a, u0b, b0)


def _upd_apply_body(h_ref, ag_ref, st_ref, u0a_ref, u0b_ref, b0_ref,
                    u1_ref, b1_ref, nl_ref, out_ref):
    j = pl.program_id(0)
    u = (jnp.dot(h_ref[...], u0a_ref[...], preferred_element_type=_f32)
         + jnp.dot(ag_ref[...], u0b_ref[...], preferred_element_type=_f32)
         + b0_ref[...])
    un = jnp.maximum((u - st_ref[0:1, :]) * st_ref[1:2, :], 0.0)
    hn = jnp.dot(un, u1_ref[...], preferred_element_type=_f32) + b1_ref[...] + nl_ref[...]
    rows = lax.broadcasted_iota(jnp.int32, (RN, 1), 0) + j * RN
    out_ref[...] = jnp.where(rows < N, hn, 0.0)


def _upd_apply(h, aggr, st, u0a, u0b, b0, u1, b1, nl):
    return pl.pallas_call(
        _upd_apply_body,
        grid=(NP // RN,),
        in_specs=[
            pl.BlockSpec((RN, LAT), lambda j: (j, 0)),
            pl.BlockSpec((RN, HID), lambda j: (j, 0)),
            pl.BlockSpec((8, HID), lambda j: (0, 0)),
            pl.BlockSpec((LAT, HID), lambda j: (0, 0)),
            pl.BlockSpec((HID, HID), lambda j: (0, 0)),
            pl.BlockSpec((1, HID), lambda j: (0, 0)),
            pl.BlockSpec((HID, LAT), lambda j: (0, 0)),
            pl.BlockSpec((1, LAT), lambda j: (0, 0)),
            pl.BlockSpec((RN, LAT), lambda j: (j, 0)),
        ],
        out_specs=pl.BlockSpec((RN, LAT), lambda j: (j, 0)),
        out_shape=jax.ShapeDtypeStruct((NP, LAT), _f32),
    )(h, aggr, st, u0a, u0b, b0, u1, b1, nl)


def _sc_pass1(eip, gf, eb):
    @functools.partial(
        pl.kernel,
        out_type=(jax.ShapeDtypeStruct((NCORE, EP, LAT), _f32),
                  jax.ShapeDtypeStruct((32, 2 * HID), _f32),
                  jax.ShapeDtypeStruct((NCORE, NP, 16), _f32)),
        mesh=_sc_mesh(),
        scratch_types=[
            pltpu.VMEM((GC,), jnp.int32),
            pltpu.VMEM((GC,), jnp.int32),
            pltpu.VMEM((GC, 2 * HID), _f32),
            pltpu.VMEM((GC, 2 * HID), _f32),
            pltpu.VMEM((GC, HID), _f32),
            pltpu.VMEM((GC, LAT), _f32),
            pltpu.VMEM((GC, LAT), _f32),
            pltpu.VMEM((2 * HID,), _f32),
            pltpu.VMEM((16, 16), _f32),
            pltpu.VMEM_SHARED((NP, 16), _f32),
            pltpu.SemaphoreType.DMA,
            pltpu.SemaphoreType.DMA,
            pltpu.SemaphoreType.DMA,
        ],
    )
    def k(ei_hbm, gf_hbm, eb_hbm, z_hbm, st_hbm, cnt_hbm,
          didx, sidx, gd, gs, ebv, zlo, zhi, svec, onesb, cacc,
          sem1, sem2, sem3):
        c = lax.axis_index("c")
        s = lax.axis_index("s")
        w = c * NTILE + s
        per_w = EP // (NCORE * NTILE)
        base = w * per_w
        zero16 = jnp.zeros((16,), _f32)
        one16 = jnp.ones((16,), _f32)
        stripe = NP // NTILE

        @pl.loop(0, 16)
        def _(r):
            onesb[r, :] = zero16

        @pl.loop(0, stripe // 16)
        def _(t):
            pltpu.sync_copy(onesb, cacc.at[pl.ds(s * stripe + t * 16, 16), :])

        @pl.loop(0, 16)
        def _(r):
            onesb[r, :] = one16

        plsc.subcore_barrier()

        def chunk_body(j, qs):
            off = base + j * GC
            pltpu.sync_copy(ei_hbm.at[1, pl.ds(off, GC)], didx)
            pltpu.sync_copy(ei_hbm.at[0, pl.ds(off, GC)], sidx)

            for kk in range(GC // 16):
                iv = didx[pl.ds(16 * kk, 16)]
                pltpu.sync_copy(onesb, cacc.at[iv], add=True)
            cp1 = pltpu.async_copy(gf_hbm.at[didx], gd, sem1)
            cp2 = pltpu.async_copy(gf_hbm.at[sidx], gs, sem2)
            cp3 = pltpu.async_copy(eb_hbm.at[pl.ds(off, GC), :], ebv, sem3)
            cp1.wait()
            cp2.wait()
            cp3.wait()

            def row_body(r, qs2):
                sums, sqs = qs2
                new_s = []
                new_q = []
                for kk in range(4):
                    zv = (gd[r, pl.ds(16 * kk, 16)]
                          + gs[r, pl.ds(HID + 16 * kk, 16)]
                          + ebv[r, pl.ds(16 * kk, 16)])
                    if kk < 2:
                        zlo[r, pl.ds(16 * kk, 16)] = zv
                    else:
                        zhi[r, pl.ds(16 * (kk - 2), 16)] = zv
                    new_s.append(sums[kk] + zv)
                    new_q.append(sqs[kk] + zv * zv)
                return (tuple(new_s), tuple(new_q))

            qs = lax.fori_loop(0, GC, row_body, qs)
            pltpu.sync_copy(zlo, z_hbm.at[0, pl.ds(off, GC), :])
            pltpu.sync_copy(zhi, z_hbm.at[1, pl.ds(off, GC), :])
            return qs

        zs4 = (zero16, zero16, zero16, zero16)
        sums, sqs = lax.fori_loop(0, per_w // GC, chunk_body, (zs4, zs4))
        for kk in range(4):
            svec[pl.ds(16 * kk, 16)] = sums[kk]
            svec[pl.ds(HID + 16 * kk, 16)] = sqs[kk]
        pltpu.sync_copy(svec, st_hbm.at[w])
        plsc.subcore_barrier()
        pltpu.sync_copy(cacc.at[pl.ds(s * stripe, stripe), :],
                        cnt_hbm.at[c, pl.ds(s * stripe, stripe), :])

    return k(eip, gf, eb)


def _sc_pass2(z, eip, mv, half):
    @functools.partial(
        pl.kernel,
        out_type=jax.ShapeDtypeStruct((NCORE, NP, 16), _f32),
        mesh=_sc_mesh(),
        scratch_types=[
            pltpu.VMEM((CHUNK, LAT), _f32),
            pltpu.VMEM((CHUNK, 16), _f32),
            pltpu.VMEM((CHUNK,), jnp.int32),
            pltpu.VMEM((8, HID), _f32),
            pltpu.VMEM_SHARED((NP, 16), _f32),
        ],
    )
    def k(z_hbm, ei_hbm, mv_hbm, out_hbm, zb, sb, idx, mvv, acc):
        c = lax.axis_index("c")
        s = lax.axis_index("s")
        pltpu.sync_copy(mv_hbm, mvv)
        cz = c == 0
        m0 = jnp.where(cz, mvv[0, pl.ds(32 * half, 16)],
                       mvv[0, pl.ds(32 * half + 16, 16)])
        zero16 = jnp.zeros((16,), _f32)

        @pl.loop(0, CHUNK)
        def _(r):
            sb[r, :] = zero16

        stripe = NP // NTILE

        @pl.loop(0, stripe // CHUNK)
        def _(t):
            pltpu.sync_copy(sb, acc.at[pl.ds(s * stripe + t * CHUNK, CHUNK), :])

        plsc.subcore_barrier()

        per_tile = EP // NTILE

        @pl.loop(0, per_tile // CHUNK)
        def _(j):
            off = s * per_tile + j * CHUNK
            pltpu.sync_copy(z_hbm.at[half, pl.ds(off, CHUNK), :], zb)
            pltpu.sync_copy(ei_hbm.at[1, pl.ds(off, CHUNK)], idx)

            @pl.loop(0, CHUNK)
            def _(r):
                lo = zb[r, pl.ds(0, 16)]
                hi = zb[r, pl.ds(16, 16)]
                sb[r, :] = jnp.maximum(jnp.where(cz, lo, hi) - m0, 0.0)

            for kk in range(8):
                iv = idx[pl.ds(16 * kk, 16)]
                pltpu.sync_copy(sb.at[pl.ds(16 * kk, 16), :],
                                acc.at[iv], add=True)

        plsc.subcore_barrier()
        pltpu.sync_copy(acc.at[pl.ds(s * stripe, stripe), :],
                        out_hbm.at[c, pl.ds(s * stripe, stripe), :])

    return k(z, eip, mv)


def _pad_rows(w, rows):
    return jnp.zeros((rows, w.shape[1]), _f32).at[:w.shape[0], :].set(w)


def kernel(x, edge_index, edge_attr, params):
    xp = jnp.zeros((NP, 8), _f32).at[:N, :3].set(x)
    eap = jnp.zeros((EP, 8), _f32).at[:E, :4].set(edge_attr)
    eip = jnp.full((2, EP), N, jnp.int32).at[:, :E].set(edge_index)

    ne = params["node_enc"]
    ee = params["edge_enc"]
    de = params["dec"]

    node_lat = _node_mlp3(
        xp,
        _pad_rows(ne["W0"], 8), ne["b0"].reshape(1, -1),
        ne["W1"], ne["b1"].reshape(1, -1),
        ne["W2"], ne["b2"].reshape(1, -1))

    est = _edge_stats(eap, _pad_rows(ee["W0"], 8), ee["b0"].reshape(1, -1))

    bcat = jnp.concatenate([params["mp"][l]["msg"]["W0"][LAT:2 * LAT]
                            for l in range(3)], axis=0)
    b0cat = _pad_rows(
        jnp.stack([params["mp"][l]["msg"]["b0"] for l in range(3)], axis=0), 8)
    eb0, eb1, eb2, _ = _edge_latent(
        eap, est, _pad_rows(ee["W0"], 8), ee["b0"].reshape(1, -1),
        ee["W1"], ee["b1"].reshape(1, -1), bcat, b0cat)
    ebs = (eb0, eb1, eb2)

    h = node_lat
    for l in range(3):
        mp = params["mp"][l]
        w0 = mp["msg"]["W0"]
        a_blk, b_blk, c_blk = w0[:LAT], w0[LAT:2 * LAT], w0[2 * LAT:]
        gf, zs, ps = _pre(h, a_blk, c_blk, b_blk,
                          mp["msg"]["b0"].reshape(1, -1))
        z = (jnp.take(gf[:, :HID], eip[1], axis=0)
             + jnp.take(gf[:, HID:], eip[0], axis=0) + ebs[l])
        scst = (jnp.zeros((32, 2 * HID), _f32)
                .at[0, :HID].set(jnp.sum(z, axis=0))
                .at[0, HID:].set(jnp.sum(z * z, axis=0)))
        mv = _fin(scst, ps)
        r = jnp.maximum(z - mv[0], 0.0)
        seg = jax.ops.segment_sum(r, eip[1], num_segments=NP)
        cntv = jax.ops.segment_sum(jnp.ones((EP,), _f32), eip[1],
                                   num_segments=NP)
        acca = jnp.stack([seg[:, :16], seg[:, 16:32]])
        accb = jnp.stack([seg[:, 32:48], seg[:, 48:]])
        cnt = jnp.stack([jnp.broadcast_to(cntv[:, None], (NP, 16)),
                         jnp.zeros((NP, 16), _f32)])
        aggr = _aggr(acca, accb, zs, mv, cnt,
                     mp["msg"]["W1"], mp["msg"]["b1"].reshape(1, -1))
        u0 = mp["upd"]["W0"]
        u0a, u0b = u0[:LAT], u0[LAT:]
        b0u = mp["upd"]["b0"].reshape(1, -1)
        ust = _upd_stats(h, aggr, u0a, u0b, b0u)
        h = _upd_apply(h, aggr, ust, u0a, u0b, b0u,
                       mp["upd"]["W1"], mp["upd"]["b1"].reshape(1, -1),
                       node_lat)

    out = _node_mlp3(
        h,
        de["W0"], de["b0"].reshape(1, -1),
        de["W1"], de["b1"].reshape(1, -1),
        jnp.zeros((HID, 8), _f32).at[:, :4].set(de["W2"]),
        jnp.zeros((1, 8), _f32).at[0, :4].set(de["b2"]))
    return out[:N, :4]
